# baseline (speedup 1.0000x reference)
import jax
import jax.numpy as jnp
from jax import lax
from jax.experimental import pallas as pl
from jax.experimental.pallas import tpu as pltpu

NBINS = 15


def _rowmax_argmax(xt, s_iota):
    v3 = xt.reshape(16, 8, 128)
    v = v3
    while v.shape[0] > 1:
        h = v.shape[0] // 2
        v = jnp.maximum(v[:h], v[h:])
    v = v[0]
    for k in (4, 2, 1):
        v = jnp.maximum(v, jnp.roll(v, k, axis=0))
    m = v3 == v[None, :, :]
    ws = [jnp.where(m[j], float(j), 3.0e4) for j in range(16)]
    while len(ws) > 1:
        h = len(ws) // 2
        ws = [jnp.minimum(ws[i], ws[i + h]) for i in range(h)]
    w = ws[0] * 8.0 + s_iota
    for k in (4, 2, 1):
        w = jnp.minimum(w, jnp.roll(w, k, axis=0))
    return v[0:1, :], w[0:1, :]


def _ece_block(x_ref, t_ref, o_ref, cnt_ref, acc_ref, cf_ref):
    i = pl.program_id(0)
    nb = pl.num_programs(0)

    @pl.when(i == 0)
    def _init():
        cnt_ref[:] = jnp.zeros_like(cnt_ref)
        acc_ref[:] = jnp.zeros_like(acc_ref)
        cf_ref[:] = jnp.zeros_like(cf_ref)

    x = x_ref[:]
    R, C = x.shape
    T = R // 128
    s_iota = lax.broadcasted_iota(jnp.int32, (8, 128), 0).astype(jnp.float32)
    confs = []
    preds = []
    for t in range(T):
        xt = x[t * 128:(t + 1) * 128, :].T
        c_t, p_t = _rowmax_argmax(xt, s_iota)
        confs.append(c_t)
        preds.append(p_t)
    conf = jnp.concatenate(confs, axis=0)
    pred = jnp.concatenate(preds, axis=0)
    tgt = t_ref[0, 0, :].reshape(T, 128).astype(jnp.float32)
    correct = (pred == tgt).astype(jnp.float32)
    binid = jnp.clip(jnp.floor(conf * NBINS).astype(jnp.int32), 0, NBINS - 1)

    ones = jnp.ones_like(conf)
    zero = jnp.zeros_like(conf)
    for b in range(NBINS):
        m = binid == b
        s = slice(b * T, (b + 1) * T)
        cnt_ref[s, :] += jnp.where(m, ones, zero)
        acc_ref[s, :] += jnp.where(m, correct, zero)
        cf_ref[s, :] += jnp.where(m, conf, zero)

    @pl.when(i == nb - 1)
    def _fin():
        n_total = nb * R
        counts = jnp.sum(cnt_ref[:].reshape(NBINS, T, 128), axis=(1, 2))
        accs = jnp.sum(acc_ref[:].reshape(NBINS, T, 128), axis=(1, 2))
        confs_ = jnp.sum(cf_ref[:].reshape(NBINS, T, 128), axis=(1, 2))
        safe = jnp.maximum(counts, 1.0)
        per_bin = jnp.where(
            counts > 0,
            (counts / n_total) * jnp.abs(accs / safe - confs_ / safe),
            0.0,
        )
        o_ref[:, :] = jnp.full((1, 128), jnp.sum(per_bin), jnp.float32)


def kernel(inputs, targets):
    N, C = inputs.shape
    R = min(4096, N)
    NB = N // R
    T = R // 128
    tgt3 = targets.astype(jnp.int32).reshape(NB, 1, R)
    out = pl.pallas_call(
        _ece_block,
        grid=(NB,),
        in_specs=[
            pl.BlockSpec((R, C), lambda i: (i, 0)),
            pl.BlockSpec((1, 1, R), lambda i: (i, 0, 0)),
        ],
        out_specs=pl.BlockSpec((1, 128), lambda i: (0, 0)),
        out_shape=jax.ShapeDtypeStruct((1, 128), jnp.float32),
        scratch_shapes=[
            pltpu.VMEM((NBINS * T, 128), jnp.float32),
            pltpu.VMEM((NBINS * T, 128), jnp.float32),
            pltpu.VMEM((NBINS * T, 128), jnp.float32),
        ],
    )(inputs, tgt3)
    return out[0, 0].reshape(())


# speedup vs baseline: 1.5044x; 1.5044x over previous
import jax
import jax.numpy as jnp
from jax import lax
from jax.experimental import pallas as pl
from jax.experimental.pallas import tpu as pltpu

NBINS = 15


def _rowmax_argmax(xt, s_iota):
    v3 = xt.reshape(16, 8, 128)
    v = v3
    while v.shape[0] > 1:
        h = v.shape[0] // 2
        v = jnp.maximum(v[:h], v[h:])
    v = v[0]
    for k in (4, 2, 1):
        v = jnp.maximum(v, jnp.roll(v, k, axis=0))
    m = v3 == v[None, :, :]
    ws = [jnp.where(m[j], float(j), 3.0e4) for j in range(16)]
    while len(ws) > 1:
        h = len(ws) // 2
        ws = [jnp.minimum(ws[i], ws[i + h]) for i in range(h)]
    w = ws[0] * 8.0 + s_iota
    for k in (4, 2, 1):
        w = jnp.minimum(w, jnp.roll(w, k, axis=0))
    return v[0:1, :], w[0:1, :]


def _ece_block(x_ref, t_ref, o_ref, cnt_ref, acc_ref, cf_ref):
    i = pl.program_id(0)
    nb = pl.num_programs(0)

    @pl.when(i == 0)
    def _init():
        cnt_ref[:] = jnp.zeros_like(cnt_ref)
        acc_ref[:] = jnp.zeros_like(acc_ref)
        cf_ref[:] = jnp.zeros_like(cf_ref)

    x = x_ref[:]
    R, C = x.shape
    T = R // 128
    s_iota = lax.broadcasted_iota(jnp.int32, (8, 128), 0).astype(jnp.float32)
    confs = []
    preds = []
    for t in range(T):
        xt = x[t * 128:(t + 1) * 128, :].T
        c_t, p_t = _rowmax_argmax(xt, s_iota)
        confs.append(c_t)
        preds.append(p_t)
    conf = jnp.concatenate(confs, axis=0)
    pred = jnp.concatenate(preds, axis=0)
    tgt = t_ref[0, 0, :].reshape(T, 128).astype(jnp.float32)
    correct = (pred == tgt).astype(jnp.float32)
    binid = jnp.clip(jnp.floor(conf * NBINS).astype(jnp.int32), 0, NBINS - 1)

    ones = jnp.ones_like(conf)
    zero = jnp.zeros_like(conf)
    for b in range(NBINS):
        m = binid == b
        s = slice(b * T, (b + 1) * T)
        cnt_ref[s, :] += jnp.where(m, ones, zero)
        acc_ref[s, :] += jnp.where(m, correct, zero)
        cf_ref[s, :] += jnp.where(m, conf, zero)

    @pl.when(i == nb - 1)
    def _fin():
        n_total = nb * R
        counts = jnp.sum(cnt_ref[:].reshape(NBINS, T, 128), axis=(1, 2))
        accs = jnp.sum(acc_ref[:].reshape(NBINS, T, 128), axis=(1, 2))
        confs_ = jnp.sum(cf_ref[:].reshape(NBINS, T, 128), axis=(1, 2))
        safe = jnp.maximum(counts, 1.0)
        per_bin = jnp.where(
            counts > 0,
            (counts / n_total) * jnp.abs(accs / safe - confs_ / safe),
            0.0,
        )
        o_ref[:, :] = jnp.full((1, 128), jnp.sum(per_bin), jnp.float32)


def kernel(inputs, targets):
    N, C = inputs.shape
    R = min(16384, N)
    NB = N // R
    T = R // 128
    tgt3 = targets.astype(jnp.int32).reshape(NB, 1, R)
    out = pl.pallas_call(
        _ece_block,
        grid=(NB,),
        in_specs=[
            pl.BlockSpec((R, C), lambda i: (i, 0)),
            pl.BlockSpec((1, 1, R), lambda i: (i, 0, 0)),
        ],
        out_specs=pl.BlockSpec((1, 128), lambda i: (0, 0)),
        out_shape=jax.ShapeDtypeStruct((1, 128), jnp.float32),
        scratch_shapes=[
            pltpu.VMEM((NBINS * T, 128), jnp.float32),
            pltpu.VMEM((NBINS * T, 128), jnp.float32),
            pltpu.VMEM((NBINS * T, 128), jnp.float32),
        ],
    )(inputs, tgt3)
    return out[0, 0].reshape(())


# speedup vs baseline: 1.5880x; 1.0556x over previous
import jax
import jax.numpy as jnp
from jax import lax
from jax.experimental import pallas as pl
from jax.experimental.pallas import tpu as pltpu

NBINS = 15


def _rowmax_argmax(xt, s_iota):
    v3 = xt.reshape(16, 8, 128)
    v = v3
    while v.shape[0] > 1:
        h = v.shape[0] // 2
        v = jnp.maximum(v[:h], v[h:])
    v = v[0]
    for k in (4, 2, 1):
        v = jnp.maximum(v, jnp.roll(v, k, axis=0))
    m = v3 == v[None, :, :]
    ws = [jnp.where(m[j], float(j), 3.0e4) for j in range(16)]
    while len(ws) > 1:
        h = len(ws) // 2
        ws = [jnp.minimum(ws[i], ws[i + h]) for i in range(h)]
    w = ws[0] * 8.0 + s_iota
    for k in (4, 2, 1):
        w = jnp.minimum(w, jnp.roll(w, k, axis=0))
    return v[0:1, :], w[0:1, :]


def _ece_block(x_ref, t_ref, o_ref, dif_ref):
    i = pl.program_id(0)
    nb = pl.num_programs(0)

    @pl.when(i == 0)
    def _init():
        dif_ref[:] = jnp.zeros_like(dif_ref)

    x = x_ref[:]
    R, C = x.shape
    T = R // 128
    s_iota = lax.broadcasted_iota(jnp.int32, (8, 128), 0).astype(jnp.float32)
    confs = []
    preds = []
    for t in range(T):
        xt = x[t * 128:(t + 1) * 128, :].T
        c_t, p_t = _rowmax_argmax(xt, s_iota)
        confs.append(c_t)
        preds.append(p_t)
    conf = jnp.concatenate(confs, axis=0)
    pred = jnp.concatenate(preds, axis=0)
    tgt = t_ref[0, 0, :].reshape(T, 128).astype(jnp.float32)
    correct = (pred == tgt).astype(jnp.float32)
    d = correct - conf
    binid = jnp.clip(jnp.floor(conf * NBINS).astype(jnp.int32), 0, NBINS - 1)

    zero = jnp.zeros_like(d)
    for b in range(NBINS):
        m = binid == b
        s = slice(b * T, (b + 1) * T)
        dif_ref[s, :] += jnp.where(m, d, zero)

    @pl.when(i == nb - 1)
    def _fin():
        n_total = nb * R
        dsum = jnp.sum(dif_ref[:].reshape(NBINS, T, 128), axis=(1, 2))
        loss = jnp.sum(jnp.abs(dsum)) / n_total
        o_ref[:, :] = jnp.full((1, 128), loss, jnp.float32)


def kernel(inputs, targets):
    N, C = inputs.shape
    R = min(16384, N)
    NB = N // R
    T = R // 128
    tgt3 = targets.astype(jnp.int32).reshape(NB, 1, R)
    out = pl.pallas_call(
        _ece_block,
        grid=(NB,),
        in_specs=[
            pl.BlockSpec((R, C), lambda i: (i, 0)),
            pl.BlockSpec((1, 1, R), lambda i: (i, 0, 0)),
        ],
        out_specs=pl.BlockSpec((1, 128), lambda i: (0, 0)),
        out_shape=jax.ShapeDtypeStruct((1, 128), jnp.float32),
        scratch_shapes=[
            pltpu.VMEM((NBINS * T, 128), jnp.float32),
        ],
    )(inputs, tgt3)
    return out[0, 0].reshape(())


# speedup vs baseline: 1.7606x; 1.1087x over previous
import jax
import jax.numpy as jnp
from jax import lax
from jax.experimental import pallas as pl
from jax.experimental.pallas import tpu as pltpu

NBINS = 15


def _rowmax_argmax(xt, s_iota):
    v3 = xt.reshape(16, 8, 128)
    v = v3
    while v.shape[0] > 1:
        h = v.shape[0] // 2
        v = jnp.maximum(v[:h], v[h:])
    v = v[0]
    for k in (4, 2, 1):
        v = jnp.maximum(v, jnp.roll(v, k, axis=0))
    m = v3 == v[None, :, :]
    ws = [jnp.where(m[j], float(j), 3.0e4) for j in range(16)]
    while len(ws) > 1:
        h = len(ws) // 2
        ws = [jnp.minimum(ws[i], ws[i + h]) for i in range(h)]
    w = ws[0] * 8.0 + s_iota
    for k in (4, 2, 1):
        w = jnp.minimum(w, jnp.roll(w, k, axis=0))
    return v[0:1, :], w[0:1, :]


def _ece_block(x_ref, t_ref, o_ref, dif_ref):
    i = pl.program_id(0)
    nb = pl.num_programs(0)

    @pl.when(i == 0)
    def _init():
        dif_ref[:] = jnp.zeros_like(dif_ref)

    x = x_ref[:]
    R, C = x.shape
    T = R // 128
    s_iota = lax.broadcasted_iota(jnp.int32, (8, 128), 0).astype(jnp.float32)
    confs = []
    preds = []
    for t in range(T):
        xt = x[t * 128:(t + 1) * 128, :].T
        c_t, p_t = _rowmax_argmax(xt, s_iota)
        confs.append(c_t)
        preds.append(p_t)
    conf = jnp.concatenate(confs, axis=0)
    pred = jnp.concatenate(preds, axis=0)
    tgt = t_ref[0, 0, :].reshape(T, 128).astype(jnp.float32)
    correct = (pred == tgt).astype(jnp.float32)
    d = correct - conf
    binid = jnp.clip(jnp.floor(conf * NBINS).astype(jnp.int32), 0, NBINS - 1)

    zero = jnp.zeros_like(d)
    for b in range(NBINS):
        m = binid == b
        s = slice(b * T, (b + 1) * T)
        dif_ref[s, :] += jnp.where(m, d, zero)

    @pl.when(i == nb - 1)
    def _fin():
        n_total = nb * R
        dsum = jnp.sum(dif_ref[:].reshape(NBINS, T, 128), axis=(1, 2))
        loss = jnp.sum(jnp.abs(dsum)) / n_total
        o_ref[:, :] = jnp.full((1, 128), loss, jnp.float32)


def kernel(inputs, targets):
    N, C = inputs.shape
    R = min(32768, N)
    NB = N // R
    T = R // 128
    tgt3 = targets.astype(jnp.int32).reshape(NB, 1, R)
    out = pl.pallas_call(
        _ece_block,
        grid=(NB,),
        in_specs=[
            pl.BlockSpec((R, C), lambda i: (i, 0)),
            pl.BlockSpec((1, 1, R), lambda i: (i, 0, 0)),
        ],
        out_specs=pl.BlockSpec((1, 128), lambda i: (0, 0)),
        out_shape=jax.ShapeDtypeStruct((1, 128), jnp.float32),
        scratch_shapes=[
            pltpu.VMEM((NBINS * T, 128), jnp.float32),
        ],
    )(inputs, tgt3)
    return out[0, 0].reshape(())
